# baseline (speedup 1.0000x reference)
import numpy as np
import jax
import jax.numpy as jnp
from jax.experimental import pallas as pl
from jax.experimental.pallas import tpu as pltpu

_EPS = 1e-6
_LAMBDA = 0.1
_GROUP = 4


def _dft_constants(H):
    HR = H // 2 + 1
    HRP = ((HR + 7) // 8) * 8
    HP = ((H + 127) // 128) * 128
    n = np.arange(H, dtype=np.int64)
    m = np.outer(n, n) % H
    ang = (-2.0 * np.pi / H) * m
    C = np.cos(ang)
    S = np.sin(ang)
    CS = np.zeros((2 * HRP, H))
    CS[:HR] = C[:HR]
    CS[HRP:HRP + HR] = S[:HR]
    W2 = np.zeros((2 * HP, 2 * H))
    W2[0:H, 0:H] = C
    W2[0:H, H:2 * H] = S
    W2[HP:HP + H, 0:H] = -S
    W2[HP:HP + H, H:2 * H] = C
    r = np.arange(HRP)
    w = np.where((r == 0) | (r == H // 2), 1.0,
                 np.where(r < H // 2, 2.0, 0.0))
    W = np.broadcast_to(w[:, None], (HRP, H)).copy()
    return (jnp.asarray(CS, jnp.bfloat16), jnp.asarray(W2, jnp.float32),
            jnp.asarray(W, jnp.float32))


def _psd_body(r_ref, cs_ref, w2_ref, psd_ref, ab2_ref):
    j = pl.program_id(1)
    hrp = psd_ref.shape[1]
    h = r_ref.shape[2]
    hp = ab2_ref.shape[2] // 2

    @pl.when(j == 0)
    def _():
        ab2_ref[...] = jnp.zeros_like(ab2_ref)

    cur = jax.lax.rem(j, 2)
    prev = jax.lax.rem(j + 1, 2)
    ab = jnp.dot(cs_ref[...], r_ref[0],
                 preferred_element_type=jnp.float32)
    reim = jnp.dot(ab2_ref[prev], w2_ref[...],
                   preferred_element_type=jnp.float32)
    contrib = reim * reim
    ab2_ref[cur, :, 0:h] = ab[0:hrp]
    ab2_ref[cur, :, hp:hp + h] = ab[hrp:2 * hrp]

    @pl.when(j == 1)
    def _():
        psd_ref[0] = contrib

    @pl.when(j > 1)
    def _():
        psd_ref[0] += contrib


def _gram_body(r_ref, gram_ref, aux_ref):
    j = pl.program_id(1)
    n, kb, w = r_ref.shape
    x = r_ref[...].reshape(n, kb * w)
    g = jax.lax.dot_general(x, x, (((1,), (1,)), ((), ())),
                            preferred_element_type=jnp.float32)
    s = jnp.sum(x, axis=1, dtype=jnp.float32)
    rows = jax.lax.broadcasted_iota(jnp.int32, (8, n), 0)
    aux = jnp.where(rows == 0, s[None, :], 0.0)

    @pl.when(j == 0)
    def _():
        gram_ref[0] = g
        aux_ref[0] = aux

    @pl.when(j > 0)
    def _():
        gram_ref[0] += g
        aux_ref[0] += aux


def kernel(r_matrix):
    n, h, w = r_matrix.shape
    return _run(r_matrix, n, h, w)


def _run(r_matrix, n, h, w):
    cs, w2, wmat = _dft_constants(h)
    hrp = cs.shape[0] // 2
    hp = w2.shape[0] // 2
    rb = r_matrix.astype(jnp.bfloat16)

    npc = n // 2
    psd = pl.pallas_call(
        _psd_body,
        grid=(2, npc + 1),
        in_specs=[
            pl.BlockSpec((1, h, w),
                         lambda i, j: (i * npc + jnp.minimum(j, npc - 1), 0, 0)),
            pl.BlockSpec((2 * hrp, w), lambda i, j: (0, 0)),
            pl.BlockSpec((2 * hp, 2 * w), lambda i, j: (0, 0)),
        ],
        out_specs=pl.BlockSpec((1, hrp, 2 * w), lambda i, j: (i, 0, 0)),
        out_shape=jax.ShapeDtypeStruct((2, hrp, 2 * w), jnp.float32),
        scratch_shapes=[pltpu.VMEM((2, hrp, 2 * hp), jnp.float32)],
        compiler_params=pltpu.CompilerParams(
            dimension_semantics=("parallel", "arbitrary")),
        name="psd_dft",
    )(rb, cs, w2)

    kb = next(b for b in (40, 32, 24, 16, 8) if (h // 2) % b == 0)
    kbn = (h // 2) // kb
    gram, aux = pl.pallas_call(
        _gram_body,
        grid=(2, kbn),
        in_specs=[pl.BlockSpec((n, kb, w), lambda i, j: (0, i * kbn + j, 0))],
        out_specs=[
            pl.BlockSpec((1, n, n), lambda i, j: (i, 0, 0)),
            pl.BlockSpec((1, 8, n), lambda i, j: (i, 0, 0)),
        ],
        out_shape=[
            jax.ShapeDtypeStruct((2, n, n), jnp.float32),
            jax.ShapeDtypeStruct((2, 8, n), jnp.float32),
        ],
        compiler_params=pltpu.CompilerParams(
            dimension_semantics=("parallel", "arbitrary"),
            vmem_limit_bytes=48 * 1024 * 1024),
        name="gram",
    )(rb)

    out = pl.pallas_call(
        _final_body,
        out_shape=jax.ShapeDtypeStruct((8, 128), jnp.float32),
        name="dist_loss_finalize",
    )(gram, aux, psd, wmat)
    return out[0, 0].reshape(())


def _final_body(gram_ref, aux_ref, psd_ref, w_ref, out_ref):
    n = gram_ref.shape[1]
    h = psd_ref.shape[2] // 2
    d = float(h * h)

    g = gram_ref[0] + gram_ref[1]
    s = aux_ref[0, 0:1, :] + aux_ref[1, 0:1, :]
    row = jax.lax.broadcasted_iota(jnp.int32, (n, n), 0)
    col = jax.lax.broadcasted_iota(jnp.int32, (n, n), 1)
    sq = jnp.sum(jnp.where(row == col, g, 0.0), axis=1)
    d2 = (sq[:, None] + sq[None, :] - 2.0 * g
          + (2.0 * _EPS) * (jnp.transpose(s) - s) + d * _EPS * _EPS)
    dist = jnp.sqrt(jnp.maximum(d2, 0.0))
    upper = col > row
    same = (row // _GROUP) == (col // _GROUP)
    s_tot = jnp.sum(jnp.where(upper, -dist, 0.0))
    p = jnp.where(upper & same, -dist, 0.0) / s_tot
    colsum = jnp.sum(p, axis=0) + jnp.sum(p, axis=1)
    loss_all = jnp.sum(-jnp.log(colsum))

    acc = psd_ref[0] + psd_ref[1]
    psd = (acc[:, 0:h] + acc[:, h:2 * h]) * jnp.float32(1.0 / n)
    wgt = w_ref[...]
    valid = wgt > 0.0
    logpsd = jnp.log(jnp.where(valid, psd, 1.0))
    mean_log = jnp.sum(wgt * logpsd) / d
    log_mean = jnp.log(jnp.sum(wgt * psd) / d)
    reg = mean_log - log_mean

    out_ref[...] = jnp.full((8, 128), loss_all - _LAMBDA * reg, jnp.float32)


# speedup vs baseline: 1.2694x; 1.2694x over previous
import numpy as np
import jax
import jax.numpy as jnp
from jax.experimental import pallas as pl
from jax.experimental.pallas import tpu as pltpu

_EPS = 1e-6
_LAMBDA = 0.1
_GROUP = 4


def _dft_constants(H):
    HR = H // 2 + 1
    HRP = ((HR + 7) // 8) * 8
    HP = ((H + 127) // 128) * 128
    n = np.arange(H, dtype=np.int64)
    m = np.outer(n, n) % H
    ang = (-2.0 * np.pi / H) * m
    C = np.cos(ang)
    S = np.sin(ang)
    CS = np.zeros((2 * HRP, H))
    CS[:HR] = C[:HR]
    CS[HRP:HRP + HR] = S[:HR]
    W2 = np.zeros((2 * HP, 2 * H))
    W2[0:H, 0:H] = C
    W2[0:H, H:2 * H] = S
    W2[HP:HP + H, 0:H] = -S
    W2[HP:HP + H, H:2 * H] = C
    r = np.arange(HRP)
    w = np.where((r == 0) | (r == H // 2), 1.0,
                 np.where(r < H // 2, 2.0, 0.0))
    W = np.broadcast_to(w[:, None], (HRP, H)).copy()
    return (jnp.asarray(CS, jnp.bfloat16), jnp.asarray(W2, jnp.float32),
            jnp.asarray(W, jnp.float32))


def _psd_body(r_ref, cs_ref, w2_ref, psd_ref, ab2_ref):
    j = pl.program_id(1)
    b_sz = r_ref.shape[0]
    hrp = psd_ref.shape[1]
    h = r_ref.shape[2]
    hp = ab2_ref.shape[2] // 2

    @pl.when(j == 0)
    def _():
        ab2_ref[...] = jnp.zeros_like(ab2_ref)

    acc = None
    for b in range(b_sz):
        ab = jnp.dot(cs_ref[...], r_ref[b],
                     preferred_element_type=jnp.float32)
        ab2_ref[b, :, 0:h] = ab[0:hrp]
        ab2_ref[b, :, hp:hp + h] = ab[hrp:2 * hrp]
        reim = jnp.dot(ab2_ref[b], w2_ref[...],
                       preferred_element_type=jnp.float32)
        c = reim * reim
        acc = c if acc is None else acc + c

    @pl.when(j == 0)
    def _():
        psd_ref[0] = acc

    @pl.when(j > 0)
    def _():
        psd_ref[0] += acc


def _gram_body(r_ref, gram_ref, aux_ref):
    j = pl.program_id(1)
    n, kb, w = r_ref.shape
    x = r_ref[...].reshape(n, kb * w)
    g = jax.lax.dot_general(x, x, (((1,), (1,)), ((), ())),
                            preferred_element_type=jnp.float32)
    s = jnp.sum(x, axis=1, dtype=jnp.float32)
    rows = jax.lax.broadcasted_iota(jnp.int32, (8, n), 0)
    aux = jnp.where(rows == 0, s[None, :], 0.0)

    @pl.when(j == 0)
    def _():
        gram_ref[0] = g
        aux_ref[0] = aux

    @pl.when(j > 0)
    def _():
        gram_ref[0] += g
        aux_ref[0] += aux


def kernel(r_matrix):
    n, h, w = r_matrix.shape
    return _run(r_matrix, n, h, w)


def _run(r_matrix, n, h, w):
    cs, w2, wmat = _dft_constants(h)
    hrp = cs.shape[0] // 2
    hp = w2.shape[0] // 2
    rb = r_matrix.astype(jnp.bfloat16)

    npc = n // 2
    bsz = next(b for b in (4, 2, 1) if npc % b == 0)
    nsteps = npc // bsz
    psd = pl.pallas_call(
        _psd_body,
        grid=(2, nsteps),
        in_specs=[
            pl.BlockSpec((bsz, h, w), lambda i, j: (i * nsteps + j, 0, 0)),
            pl.BlockSpec((2 * hrp, w), lambda i, j: (0, 0)),
            pl.BlockSpec((2 * hp, 2 * w), lambda i, j: (0, 0)),
        ],
        out_specs=pl.BlockSpec((1, hrp, 2 * w), lambda i, j: (i, 0, 0)),
        out_shape=jax.ShapeDtypeStruct((2, hrp, 2 * w), jnp.float32),
        scratch_shapes=[pltpu.VMEM((bsz, hrp, 2 * hp), jnp.float32)],
        compiler_params=pltpu.CompilerParams(
            dimension_semantics=("parallel", "arbitrary")),
        name="psd_dft",
    )(rb, cs, w2)

    kb = next(b for b in (40, 32, 24, 16, 8) if (h // 2) % b == 0)
    kbn = (h // 2) // kb
    gram, aux = pl.pallas_call(
        _gram_body,
        grid=(2, kbn),
        in_specs=[pl.BlockSpec((n, kb, w), lambda i, j: (0, i * kbn + j, 0))],
        out_specs=[
            pl.BlockSpec((1, n, n), lambda i, j: (i, 0, 0)),
            pl.BlockSpec((1, 8, n), lambda i, j: (i, 0, 0)),
        ],
        out_shape=[
            jax.ShapeDtypeStruct((2, n, n), jnp.float32),
            jax.ShapeDtypeStruct((2, 8, n), jnp.float32),
        ],
        compiler_params=pltpu.CompilerParams(
            dimension_semantics=("parallel", "arbitrary"),
            vmem_limit_bytes=48 * 1024 * 1024),
        name="gram",
    )(rb)

    out = pl.pallas_call(
        _final_body,
        out_shape=jax.ShapeDtypeStruct((8, 128), jnp.float32),
        name="dist_loss_finalize",
    )(gram, aux, psd, wmat)
    return out[0, 0].reshape(())


def _final_body(gram_ref, aux_ref, psd_ref, w_ref, out_ref):
    n = gram_ref.shape[1]
    h = psd_ref.shape[2] // 2
    d = float(h * h)

    g = gram_ref[0] + gram_ref[1]
    s = aux_ref[0, 0:1, :] + aux_ref[1, 0:1, :]
    row = jax.lax.broadcasted_iota(jnp.int32, (n, n), 0)
    col = jax.lax.broadcasted_iota(jnp.int32, (n, n), 1)
    sq = jnp.sum(jnp.where(row == col, g, 0.0), axis=1)
    d2 = (sq[:, None] + sq[None, :] - 2.0 * g
          + (2.0 * _EPS) * (jnp.transpose(s) - s) + d * _EPS * _EPS)
    dist = jnp.sqrt(jnp.maximum(d2, 0.0))
    upper = col > row
    same = (row // _GROUP) == (col // _GROUP)
    s_tot = jnp.sum(jnp.where(upper, -dist, 0.0))
    p = jnp.where(upper & same, -dist, 0.0) / s_tot
    colsum = jnp.sum(p, axis=0) + jnp.sum(p, axis=1)
    loss_all = jnp.sum(-jnp.log(colsum))

    acc = psd_ref[0] + psd_ref[1]
    psd = (acc[:, 0:h] + acc[:, h:2 * h]) * jnp.float32(1.0 / n)
    wgt = w_ref[...]
    valid = wgt > 0.0
    logpsd = jnp.log(jnp.where(valid, psd, 1.0))
    mean_log = jnp.sum(wgt * logpsd) / d
    log_mean = jnp.log(jnp.sum(wgt * psd) / d)
    reg = mean_log - log_mean

    out_ref[...] = jnp.full((8, 128), loss_all - _LAMBDA * reg, jnp.float32)


# speedup vs baseline: 1.3046x; 1.0277x over previous
import numpy as np
import jax
import jax.numpy as jnp
from jax.experimental import pallas as pl
from jax.experimental.pallas import tpu as pltpu

_EPS = 1e-6
_LAMBDA = 0.1
_GROUP = 4


def _dft_constants(H):
    HR = H // 2 + 1
    HRP = ((HR + 7) // 8) * 8
    HP = ((H + 127) // 128) * 128
    n = np.arange(H, dtype=np.int64)
    m = np.outer(n, n) % H
    ang = (-2.0 * np.pi / H) * m
    C = np.cos(ang)
    S = np.sin(ang)
    CS = np.zeros((2 * HRP, H))
    CS[:HR] = C[:HR]
    CS[HRP:HRP + HR] = S[:HR]
    W2 = np.zeros((2 * HP, 2 * H))
    W2[0:H, 0:H] = C
    W2[0:H, H:2 * H] = S
    W2[HP:HP + H, 0:H] = -S
    W2[HP:HP + H, H:2 * H] = C
    r = np.arange(HRP)
    w = np.where((r == 0) | (r == H // 2), 1.0,
                 np.where(r < H // 2, 2.0, 0.0))
    W = np.broadcast_to(w[:, None], (HRP, H)).copy()
    return (jnp.asarray(CS, jnp.bfloat16), jnp.asarray(W2, jnp.float32),
            jnp.asarray(W, jnp.float32))


def _psd_body(r_ref, cs_ref, w2_ref, psd_ref, ab2_ref):
    j = pl.program_id(1)
    b_sz = r_ref.shape[0]
    hrp = psd_ref.shape[1]
    h = r_ref.shape[2]
    hp = ab2_ref.shape[2] // 2

    @pl.when(j == 0)
    def _():
        ab2_ref[...] = jnp.zeros_like(ab2_ref)

    acc = None
    for b in range(b_sz):
        ab = jnp.dot(cs_ref[...], r_ref[b],
                     preferred_element_type=jnp.float32)
        ab2_ref[b, :, 0:h] = ab[0:hrp]
        ab2_ref[b, :, hp:hp + h] = ab[hrp:2 * hrp]
        reim = jnp.dot(ab2_ref[b], w2_ref[...],
                       preferred_element_type=jnp.float32)
        c = reim * reim
        acc = c if acc is None else acc + c

    @pl.when(j == 0)
    def _():
        psd_ref[0] = acc

    @pl.when(j > 0)
    def _():
        psd_ref[0] += acc


def _gram_body(r_ref, gram_ref, aux_ref):
    j = pl.program_id(1)
    n, kb, w = r_ref.shape
    x = r_ref[...].reshape(n, kb * w)
    g = jax.lax.dot_general(x, x, (((1,), (1,)), ((), ())),
                            preferred_element_type=jnp.float32)
    s = jnp.sum(x, axis=1, dtype=jnp.float32)
    rows = jax.lax.broadcasted_iota(jnp.int32, (8, n), 0)
    aux = jnp.where(rows == 0, s[None, :], 0.0)

    @pl.when(j == 0)
    def _():
        gram_ref[0] = g
        aux_ref[0] = aux

    @pl.when(j > 0)
    def _():
        gram_ref[0] += g
        aux_ref[0] += aux


def kernel(r_matrix):
    n, h, w = r_matrix.shape
    return _run(r_matrix, n, h, w)


def _run(r_matrix, n, h, w):
    cs, w2, wmat = _dft_constants(h)
    hrp = cs.shape[0] // 2
    hp = w2.shape[0] // 2
    rb = r_matrix.astype(jnp.bfloat16)

    npc = n // 2
    bsz = next(b for b in (8, 4, 2, 1) if npc % b == 0)
    nsteps = npc // bsz
    psd = pl.pallas_call(
        _psd_body,
        grid=(2, nsteps),
        in_specs=[
            pl.BlockSpec((bsz, h, w), lambda i, j: (i * nsteps + j, 0, 0)),
            pl.BlockSpec((2 * hrp, w), lambda i, j: (0, 0)),
            pl.BlockSpec((2 * hp, 2 * w), lambda i, j: (0, 0)),
        ],
        out_specs=pl.BlockSpec((1, hrp, 2 * w), lambda i, j: (i, 0, 0)),
        out_shape=jax.ShapeDtypeStruct((2, hrp, 2 * w), jnp.float32),
        scratch_shapes=[pltpu.VMEM((bsz, hrp, 2 * hp), jnp.float32)],
        compiler_params=pltpu.CompilerParams(
            dimension_semantics=("parallel", "arbitrary")),
        name="psd_dft",
    )(rb, cs, w2)

    kb = next(b for b in (40, 32, 24, 16, 8) if (h // 2) % b == 0)
    kbn = (h // 2) // kb
    gram, aux = pl.pallas_call(
        _gram_body,
        grid=(2, kbn),
        in_specs=[pl.BlockSpec((n, kb, w), lambda i, j: (0, i * kbn + j, 0))],
        out_specs=[
            pl.BlockSpec((1, n, n), lambda i, j: (i, 0, 0)),
            pl.BlockSpec((1, 8, n), lambda i, j: (i, 0, 0)),
        ],
        out_shape=[
            jax.ShapeDtypeStruct((2, n, n), jnp.float32),
            jax.ShapeDtypeStruct((2, 8, n), jnp.float32),
        ],
        compiler_params=pltpu.CompilerParams(
            dimension_semantics=("parallel", "arbitrary"),
            vmem_limit_bytes=48 * 1024 * 1024),
        name="gram",
    )(rb)

    out = pl.pallas_call(
        _final_body,
        out_shape=jax.ShapeDtypeStruct((8, 128), jnp.float32),
        name="dist_loss_finalize",
    )(gram, aux, psd, wmat)
    return out[0, 0].reshape(())


def _final_body(gram_ref, aux_ref, psd_ref, w_ref, out_ref):
    n = gram_ref.shape[1]
    h = psd_ref.shape[2] // 2
    d = float(h * h)

    g = gram_ref[0] + gram_ref[1]
    s = aux_ref[0, 0:1, :] + aux_ref[1, 0:1, :]
    row = jax.lax.broadcasted_iota(jnp.int32, (n, n), 0)
    col = jax.lax.broadcasted_iota(jnp.int32, (n, n), 1)
    sq = jnp.sum(jnp.where(row == col, g, 0.0), axis=1)
    d2 = (sq[:, None] + sq[None, :] - 2.0 * g
          + (2.0 * _EPS) * (jnp.transpose(s) - s) + d * _EPS * _EPS)
    dist = jnp.sqrt(jnp.maximum(d2, 0.0))
    upper = col > row
    same = (row // _GROUP) == (col // _GROUP)
    s_tot = jnp.sum(jnp.where(upper, -dist, 0.0))
    p = jnp.where(upper & same, -dist, 0.0) / s_tot
    colsum = jnp.sum(p, axis=0) + jnp.sum(p, axis=1)
    loss_all = jnp.sum(-jnp.log(colsum))

    acc = psd_ref[0] + psd_ref[1]
    psd = (acc[:, 0:h] + acc[:, h:2 * h]) * jnp.float32(1.0 / n)
    wgt = w_ref[...]
    valid = wgt > 0.0
    logpsd = jnp.log(jnp.where(valid, psd, 1.0))
    mean_log = jnp.sum(wgt * logpsd) / d
    log_mean = jnp.log(jnp.sum(wgt * psd) / d)
    reg = mean_log - log_mean

    out_ref[...] = jnp.full((8, 128), loss_all - _LAMBDA * reg, jnp.float32)
